# baseline (speedup 1.0000x reference)
import functools

import jax
import jax.numpy as jnp
from jax import lax
from jax.experimental import pallas as pl
from jax.experimental.pallas import tpu as pltpu
from jax.experimental.pallas import tpu_sc as plsc


def _iota16(offset=0):
    return lax.iota(jnp.int32, 16) + offset


@functools.lru_cache(maxsize=None)
def _build(batch: int, seq: int, vocab: int, dim: int):
    info = plsc.get_sparse_core_info()
    nc = info.num_cores
    nw = nc * info.num_subcores
    assert dim == 64 and batch % (128 * nw // 32) == 0

    vp = vocab // 2
    nblk = vocab // 128
    tail = vocab - nblk * 128
    kmax = -(-nblk // nw)
    nb_blk = batch // 128
    assert nb_blk == nw and seq % 2 == 0 and tail % 2 == 0

    mesh = plsc.VectorSubcoreMesh(core_axis_name="c", subcore_axis_name="s")
    cp = pltpu.CompilerParams(use_tc_tiling_on_sc=True, needs_layout_passes=False)

    @functools.partial(
        pl.kernel,
        mesh=mesh,
        out_type=jax.ShapeDtypeStruct((vp, 128), jnp.float32),
        scratch_types=[
            pltpu.VMEM((2, 64, 128), jnp.float32),
            pltpu.VMEM((2, 64, 128), jnp.float32),
            pltpu.VMEM((32, 128), jnp.float32),
            pltpu.SemaphoreType.DMA,
            pltpu.SemaphoreType.DMA,
            pltpu.SemaphoreType.DMA,
            pltpu.SemaphoreType.DMA,
            pltpu.SemaphoreType.DMA,
        ],
        compiler_params=cp,
    )
    def relayout_kernel(tt_hbm, tail_hbm, pt_hbm, src_v, dst_v, tl_v,
                        si0, si1, so0, so1, st):
        wid = lax.axis_index("s") * nc + lax.axis_index("c")
        si = (si0, si1)
        so = (so0, so1)

        def ld(k, b):
            return pltpu.make_async_copy(
                tt_hbm.at[:, pl.ds((wid + nw * k) * 128, 128)], src_v.at[b], si[b]
            )

        def sto(k, b):
            return pltpu.make_async_copy(
                dst_v.at[b], pt_hbm.at[pl.ds((wid + nw * k) * 64, 64)], so[b]
            )

        rows = [_iota16(16 * eg) for eg in range(4)]

        def assemble(b):
            for j in range(64):
                for h in range(2):
                    col = jnp.full((16,), 2 * j + h, jnp.int32)
                    for eg in range(4):
                        v = plsc.load_gather(src_v.at[b], [rows[eg], col])
                        dst_v[b, j, pl.ds(64 * h + 16 * eg, 16)] = v

        @pl.when(wid == 0)
        def _():
            pltpu.sync_copy(tail_hbm, tl_v)
            pltpu.sync_copy(tl_v, pt_hbm.at[pl.ds(nblk * 64, tail // 2)])

        ld(0, 0).start()

        def body(k2, carry):
            for b in (0, 1):
                k = 2 * k2 + b

                @pl.when(wid + nw * k < nblk)
                def _():
                    ld(k, b).wait()
                    assemble(b)

                    @pl.when(k > 1)
                    def _():
                        sto(k - 2, b).wait()

                    sto(k, b).start()

                @pl.when(wid + nw * (k + 1) < nblk)
                def _():
                    ld(k + 1, 1 - b).start()

            return carry

        lax.fori_loop(0, (kmax + 1) // 2, body, 0)

        @pl.when(wid + nw * (kmax - 2) < nblk)
        def _():
            sto(kmax - 2, (kmax - 2) % 2).wait()

        @pl.when(wid + nw * (kmax - 1) < nblk)
        def _():
            sto(kmax - 1, (kmax - 1) % 2).wait()

    @functools.partial(
        pl.kernel,
        mesh=mesh,
        out_type=jax.ShapeDtypeStruct((seq, dim, batch), jnp.float32),
        scratch_types=[
            pltpu.VMEM((seq, 128), jnp.int32),
            pltpu.VMEM((seq, 128), jnp.int32),
            pltpu.VMEM((2, 128, 128), jnp.float32),
            pltpu.VMEM((2, dim, 128), jnp.float32),
            pltpu.SemaphoreType.DMA,
            pltpu.SemaphoreType.DMA,
            pltpu.SemaphoreType.DMA,
            pltpu.SemaphoreType.DMA,
            pltpu.SemaphoreType.DMA,
        ],
        compiler_params=cp,
    )
    def gather_kernel(xt_hbm, pt_hbm, out_hbm, x_v, ix_v, g_v, sl_v,
                      sg0, sg1, so0, so1, sx):
        wid = lax.axis_index("s") * nc + lax.axis_index("c")
        b0 = wid * 128
        sg = (sg0, sg1)
        so = (so0, so1)

        pltpu.sync_copy(xt_hbm.at[:, pl.ds(b0, 128)], x_v)

        def shift_body(i, carry):
            for g in range(8):
                ix_v[i, pl.ds(16 * g, 16)] = (
                    x_v[i, pl.ds(16 * g, 16)] >> 1
                )
            return carry

        lax.fori_loop(0, seq, shift_body, 0)

        def gat(s, b):
            return pltpu.make_async_copy(
                pt_hbm.at[ix_v.at[s]], g_v.at[b], sg[b]
            )

        def sto(s, b):
            return pltpu.make_async_copy(
                sl_v.at[b], out_hbm.at[s, :, pl.ds(b0, 128)], so[b]
            )

        rows = [_iota16(16 * g) for g in range(8)]

        def assemble(s, b):
            for g in range(8):
                odd = (x_v[s, pl.ds(16 * g, 16)] & 1) * 64
                for e in range(dim):
                    v = plsc.load_gather(g_v.at[b], [rows[g], odd + e])
                    sl_v[b, e, pl.ds(16 * g, 16)] = v

        gat(0, 0).start()

        def body(s2, carry):
            for b in (0, 1):
                s = 2 * s2 + b
                gat(s, b).wait()

                @pl.when(s < seq - 1)
                def _():
                    gat(s + 1, 1 - b).start()

                @pl.when(s > 1)
                def _():
                    sto(s - 2, b).wait()

                assemble(s, b)
                sto(s, b).start()
            return carry

        lax.fori_loop(0, seq // 2, body, 0)
        sto(seq - 2, 0).wait()
        sto(seq - 1, 1).wait()

    return relayout_kernel, gather_kernel


def kernel(x, table):
    batch, seq = x.shape
    vocab, dim = table.shape
    k1, k2 = _build(batch, seq, vocab, dim)
    nblk = vocab // 128
    tail2 = table[nblk * 128:].reshape(-1, 2 * dim)
    paired = k1(table.T, tail2)
    out = k2(x.T.astype(jnp.int32), paired)
    return jnp.transpose(out, (2, 0, 1))


# speedup vs baseline: 1.5238x; 1.5238x over previous
import functools

import jax
import jax.numpy as jnp
from jax import lax
from jax.experimental import pallas as pl
from jax.experimental.pallas import tpu as pltpu
from jax.experimental.pallas import tpu_sc as plsc


def _iota16(offset=0):
    return lax.iota(jnp.int32, 16) + offset


@functools.lru_cache(maxsize=None)
def _build(batch: int, seq: int, vocab: int, dim: int):
    info = plsc.get_sparse_core_info()
    nc = info.num_cores
    nw = nc * info.num_subcores
    assert dim == 64 and batch % (128 * nw // 32) == 0

    vp = vocab // 2
    nblk = vocab // 128
    tail = vocab - nblk * 128
    kmax = -(-nblk // nw)
    nb_blk = batch // 128
    assert nb_blk == nw and seq % 2 == 0 and tail % 2 == 0

    mesh = plsc.VectorSubcoreMesh(core_axis_name="c", subcore_axis_name="s")
    cp = pltpu.CompilerParams(use_tc_tiling_on_sc=True, needs_layout_passes=False)

    @functools.partial(
        pl.kernel,
        mesh=mesh,
        out_type=jax.ShapeDtypeStruct((vp, 128), jnp.float32),
        scratch_types=[
            pltpu.VMEM((2, 64, 128), jnp.float32),
            pltpu.VMEM((2, 64, 128), jnp.float32),
            pltpu.VMEM((32, 128), jnp.float32),
            pltpu.SemaphoreType.DMA,
            pltpu.SemaphoreType.DMA,
            pltpu.SemaphoreType.DMA,
            pltpu.SemaphoreType.DMA,
            pltpu.SemaphoreType.DMA,
        ],
        compiler_params=cp,
    )
    def relayout_kernel(tt_hbm, tail_hbm, pt_hbm, src_v, dst_v, tl_v,
                        si0, si1, so0, so1, st):
        wid = lax.axis_index("s") * nc + lax.axis_index("c")
        si = (si0, si1)
        so = (so0, so1)

        def ld(k, b):
            return pltpu.make_async_copy(
                tt_hbm.at[:, pl.ds((wid + nw * k) * 128, 128)], src_v.at[b], si[b]
            )

        def sto(k, b):
            return pltpu.make_async_copy(
                dst_v.at[b], pt_hbm.at[pl.ds((wid + nw * k) * 64, 64)], so[b]
            )

        rows = [_iota16(16 * eg) for eg in range(4)]

        def assemble(b):
            ops = [(j, h, eg) for j in range(64) for h in range(2)
                   for eg in range(4)]
            depth = 8
            vals = {}
            for i, (j, h, eg) in enumerate(ops):
                col = jnp.full((16,), 2 * j + h, jnp.int32)
                vals[i] = (j, h, eg,
                           plsc.load_gather(src_v.at[b], [rows[eg], col]))
                if i >= depth:
                    jj, hh, ee, v = vals.pop(i - depth)
                    dst_v[b, jj, pl.ds(64 * hh + 16 * ee, 16)] = v
            for i in sorted(vals):
                jj, hh, ee, v = vals[i]
                dst_v[b, jj, pl.ds(64 * hh + 16 * ee, 16)] = v

        @pl.when(wid == 0)
        def _():
            pltpu.sync_copy(tail_hbm, tl_v)
            pltpu.sync_copy(tl_v, pt_hbm.at[pl.ds(nblk * 64, tail // 2)])

        ld(0, 0).start()

        def body(k2, carry):
            for b in (0, 1):
                k = 2 * k2 + b

                @pl.when(wid + nw * k < nblk)
                def _():
                    ld(k, b).wait()
                    assemble(b)

                    @pl.when(k > 1)
                    def _():
                        sto(k - 2, b).wait()

                    sto(k, b).start()

                @pl.when(wid + nw * (k + 1) < nblk)
                def _():
                    ld(k + 1, 1 - b).start()

            return carry

        lax.fori_loop(0, (kmax + 1) // 2, body, 0)

        @pl.when(wid + nw * (kmax - 2) < nblk)
        def _():
            sto(kmax - 2, (kmax - 2) % 2).wait()

        @pl.when(wid + nw * (kmax - 1) < nblk)
        def _():
            sto(kmax - 1, (kmax - 1) % 2).wait()

    @functools.partial(
        pl.kernel,
        mesh=mesh,
        out_type=jax.ShapeDtypeStruct((seq, dim, batch), jnp.float32),
        scratch_types=[
            pltpu.VMEM((seq, 128), jnp.int32),
            pltpu.VMEM((seq, 128), jnp.int32),
            pltpu.VMEM((2, 128, 128), jnp.float32),
            pltpu.VMEM((2, dim, 128), jnp.float32),
            pltpu.SemaphoreType.DMA,
            pltpu.SemaphoreType.DMA,
            pltpu.SemaphoreType.DMA,
            pltpu.SemaphoreType.DMA,
            pltpu.SemaphoreType.DMA,
        ],
        compiler_params=cp,
    )
    def gather_kernel(xt_hbm, pt_hbm, out_hbm, x_v, ix_v, g_v, sl_v,
                      sg0, sg1, so0, so1, sx):
        wid = lax.axis_index("s") * nc + lax.axis_index("c")
        b0 = wid * 128
        sg = (sg0, sg1)
        so = (so0, so1)

        pltpu.sync_copy(xt_hbm.at[:, pl.ds(b0, 128)], x_v)

        def shift_body(i, carry):
            for g in range(8):
                ix_v[i, pl.ds(16 * g, 16)] = (
                    x_v[i, pl.ds(16 * g, 16)] >> 1
                )
            return carry

        lax.fori_loop(0, seq, shift_body, 0)

        def gat(s, b):
            return pltpu.make_async_copy(
                pt_hbm.at[ix_v.at[s]], g_v.at[b], sg[b]
            )

        def sto(s, b):
            return pltpu.make_async_copy(
                sl_v.at[b], out_hbm.at[s, :, pl.ds(b0, 128)], so[b]
            )

        rows = [_iota16(16 * g) for g in range(8)]

        def assemble(s, b):
            odds = [(x_v[s, pl.ds(16 * g, 16)] & 1) * 64 for g in range(8)]
            ops = [(g, e) for g in range(8) for e in range(dim)]
            depth = 8
            vals = {}
            for i, (g, e) in enumerate(ops):
                vals[i] = (g, e,
                           plsc.load_gather(g_v.at[b], [rows[g], odds[g] + e]))
                if i >= depth:
                    gg, ee, v = vals.pop(i - depth)
                    sl_v[b, ee, pl.ds(16 * gg, 16)] = v
            for i in sorted(vals):
                gg, ee, v = vals[i]
                sl_v[b, ee, pl.ds(16 * gg, 16)] = v

        gat(0, 0).start()

        def body(s2, carry):
            for b in (0, 1):
                s = 2 * s2 + b
                gat(s, b).wait()

                @pl.when(s < seq - 1)
                def _():
                    gat(s + 1, 1 - b).start()

                @pl.when(s > 1)
                def _():
                    sto(s - 2, b).wait()

                assemble(s, b)
                sto(s, b).start()
            return carry

        lax.fori_loop(0, seq // 2, body, 0)
        sto(seq - 2, 0).wait()
        sto(seq - 1, 1).wait()

    return relayout_kernel, gather_kernel


def kernel(x, table):
    batch, seq = x.shape
    vocab, dim = table.shape
    k1, k2 = _build(batch, seq, vocab, dim)
    nblk = vocab // 128
    tail2 = table[nblk * 128:].reshape(-1, 2 * dim)
    paired = k1(table.T, tail2)
    out = k2(x.T.astype(jnp.int32), paired)
    return jnp.transpose(out, (2, 0, 1))


# speedup vs baseline: 1.7315x; 1.1363x over previous
import functools

import jax
import jax.numpy as jnp
from jax import lax
from jax.experimental import pallas as pl
from jax.experimental.pallas import tpu as pltpu
from jax.experimental.pallas import tpu_sc as plsc


def _iota16(offset=0):
    return lax.iota(jnp.int32, 16) + offset


@functools.lru_cache(maxsize=None)
def _build(batch: int, seq: int, vocab: int, dim: int):
    info = plsc.get_sparse_core_info()
    nc = info.num_cores
    nw = nc * info.num_subcores
    assert dim == 64 and batch % (128 * nw // 32) == 0

    vp = vocab // 2
    nblk = vocab // 128
    tail = vocab - nblk * 128
    kmax = -(-nblk // nw)
    nb_blk = batch // 128
    assert nb_blk == nw and seq % 2 == 0 and tail % 2 == 0

    mesh = plsc.VectorSubcoreMesh(core_axis_name="c", subcore_axis_name="s")
    cp = pltpu.CompilerParams(use_tc_tiling_on_sc=True, needs_layout_passes=False)

    @functools.partial(
        pl.kernel,
        mesh=mesh,
        out_type=jax.ShapeDtypeStruct((vp, 128), jnp.float32),
        scratch_types=[
            pltpu.VMEM((2, 64, 128), jnp.float32),
            pltpu.VMEM((2, 64, 128), jnp.float32),
            pltpu.VMEM((32, 128), jnp.float32),
            pltpu.SemaphoreType.DMA,
            pltpu.SemaphoreType.DMA,
            pltpu.SemaphoreType.DMA,
            pltpu.SemaphoreType.DMA,
            pltpu.SemaphoreType.DMA,
        ],
        compiler_params=cp,
    )
    def relayout_kernel(tt_hbm, tail_hbm, pt_hbm, src_v, dst_v, tl_v,
                        si0, si1, so0, so1, st):
        wid = lax.axis_index("s") * nc + lax.axis_index("c")
        si = (si0, si1)
        so = (so0, so1)

        def ld(k, b):
            return pltpu.make_async_copy(
                tt_hbm.at[:, pl.ds((wid + nw * k) * 128, 128)], src_v.at[b], si[b]
            )

        def sto(k, b):
            return pltpu.make_async_copy(
                dst_v.at[b], pt_hbm.at[pl.ds((wid + nw * k) * 64, 64)], so[b]
            )

        rows = [_iota16(16 * eg) for eg in range(4)]

        def assemble(b):
            ops = [(j, h, eg) for j in range(64) for h in range(2)
                   for eg in range(4)]
            depth = 8
            vals = {}
            for i, (j, h, eg) in enumerate(ops):
                col = jnp.full((16,), 2 * j + h, jnp.int32)
                vals[i] = (j, h, eg,
                           plsc.load_gather(src_v.at[b], [rows[eg], col]))
                if i >= depth:
                    jj, hh, ee, v = vals.pop(i - depth)
                    dst_v[b, jj, pl.ds(64 * hh + 16 * ee, 16)] = v
            for i in sorted(vals):
                jj, hh, ee, v = vals[i]
                dst_v[b, jj, pl.ds(64 * hh + 16 * ee, 16)] = v

        @pl.when(wid == 0)
        def _():
            pltpu.sync_copy(tail_hbm, tl_v)
            pltpu.sync_copy(tl_v, pt_hbm.at[pl.ds(nblk * 64, tail // 2)])

        ld(0, 0).start()

        def body(k2, carry):
            for b in (0, 1):
                k = 2 * k2 + b

                @pl.when(wid + nw * k < nblk)
                def _():
                    ld(k, b).wait()

                    @pl.when(wid + nw * (k + 1) < nblk)
                    def _():
                        ld(k + 1, 1 - b).start()

                    assemble(b)

                    @pl.when(k > 1)
                    def _():
                        sto(k - 2, b).wait()

                    sto(k, b).start()

            return carry

        lax.fori_loop(0, (kmax + 1) // 2, body, 0)

        @pl.when(wid + nw * (kmax - 2) < nblk)
        def _():
            sto(kmax - 2, (kmax - 2) % 2).wait()

        @pl.when(wid + nw * (kmax - 1) < nblk)
        def _():
            sto(kmax - 1, (kmax - 1) % 2).wait()

    @functools.partial(
        pl.kernel,
        mesh=mesh,
        out_type=jax.ShapeDtypeStruct((seq, dim, batch), jnp.float32),
        scratch_types=[
            pltpu.VMEM((seq, 128), jnp.int32),
            pltpu.VMEM((seq, 128), jnp.int32),
            pltpu.VMEM((2, 128, 128), jnp.float32),
            pltpu.VMEM((2, dim, 128), jnp.float32),
            pltpu.SemaphoreType.DMA,
            pltpu.SemaphoreType.DMA,
            pltpu.SemaphoreType.DMA,
            pltpu.SemaphoreType.DMA,
            pltpu.SemaphoreType.DMA,
        ],
        compiler_params=cp,
    )
    def gather_kernel(xt_hbm, pt_hbm, out_hbm, x_v, ix_v, g_v, sl_v,
                      sg0, sg1, so0, so1, sx):
        wid = lax.axis_index("s") * nc + lax.axis_index("c")
        b0 = wid * 128
        sg = (sg0, sg1)
        so = (so0, so1)

        pltpu.sync_copy(xt_hbm.at[:, pl.ds(b0, 128)], x_v)

        def shift_body(i, carry):
            for g in range(8):
                ix_v[i, pl.ds(16 * g, 16)] = (
                    x_v[i, pl.ds(16 * g, 16)] >> 1
                )
            return carry

        lax.fori_loop(0, seq, shift_body, 0)

        def gat(s, b):
            return pltpu.make_async_copy(
                pt_hbm.at[ix_v.at[s]], g_v.at[b], sg[b]
            )

        def sto(s, b):
            return pltpu.make_async_copy(
                sl_v.at[b], out_hbm.at[s, :, pl.ds(b0, 128)], so[b]
            )

        rows = [_iota16(16 * g) for g in range(8)]

        def assemble(s, b):
            odds = [(x_v[s, pl.ds(16 * g, 16)] & 1) * 64 for g in range(8)]
            ops = [(g, e) for g in range(8) for e in range(dim)]
            depth = 8
            vals = {}
            for i, (g, e) in enumerate(ops):
                vals[i] = (g, e,
                           plsc.load_gather(g_v.at[b], [rows[g], odds[g] + e]))
                if i >= depth:
                    gg, ee, v = vals.pop(i - depth)
                    sl_v[b, ee, pl.ds(16 * gg, 16)] = v
            for i in sorted(vals):
                gg, ee, v = vals[i]
                sl_v[b, ee, pl.ds(16 * gg, 16)] = v

        gat(0, 0).start()

        def body(s2, carry):
            for b in (0, 1):
                s = 2 * s2 + b
                gat(s, b).wait()

                @pl.when(s < seq - 1)
                def _():
                    gat(s + 1, 1 - b).start()

                @pl.when(s > 1)
                def _():
                    sto(s - 2, b).wait()

                assemble(s, b)
                sto(s, b).start()
            return carry

        lax.fori_loop(0, seq // 2, body, 0)
        sto(seq - 2, 0).wait()
        sto(seq - 1, 1).wait()

    return relayout_kernel, gather_kernel


def kernel(x, table):
    batch, seq = x.shape
    vocab, dim = table.shape
    k1, k2 = _build(batch, seq, vocab, dim)
    nblk = vocab // 128
    tail2 = table[nblk * 128:].reshape(-1, 2 * dim)
    paired = k1(table.T, tail2)
    out = k2(x.T.astype(jnp.int32), paired)
    return jnp.transpose(out, (2, 0, 1))


# speedup vs baseline: 2.5124x; 1.4510x over previous
import functools

import jax
import jax.numpy as jnp
from jax import lax
from jax.experimental import pallas as pl
from jax.experimental.pallas import tpu as pltpu
from jax.experimental.pallas import tpu_sc as plsc

CHUNK = 800


@functools.lru_cache(maxsize=None)
def _build(n_rows: int, vocab: int, dim: int):
    info = plsc.get_sparse_core_info()
    nw = info.num_cores * info.num_subcores
    per_w = n_rows // nw
    assert n_rows % nw == 0 and per_w % CHUNK == 0 and per_w % 8 == 0
    n_chunks = per_w // CHUNK
    assert n_chunks % 2 == 0
    n2 = n_chunks // 2

    mesh = plsc.VectorSubcoreMesh(core_axis_name="c", subcore_axis_name="s")

    @functools.partial(
        pl.kernel,
        mesh=mesh,
        out_type=jax.ShapeDtypeStruct((n_rows, dim), jnp.float32),
        scratch_types=[
            pltpu.VMEM((per_w,), jnp.int32),
            pltpu.VMEM((2, CHUNK, dim), jnp.float32),
            pltpu.SemaphoreType.DMA,
            pltpu.SemaphoreType.DMA,
            pltpu.SemaphoreType.DMA,
            pltpu.SemaphoreType.DMA,
        ],
        compiler_params=pltpu.CompilerParams(use_tc_tiling_on_sc=False),
    )
    def gather_kernel(x_hbm, table_hbm, out_hbm, idx_v, rows_v, sg0, sg1, so0, so1):
        wid = lax.axis_index("s") * info.num_cores + lax.axis_index("c")
        base = wid * per_w
        sg = (sg0, sg1)
        so = (so0, so1)

        pltpu.sync_copy(x_hbm.at[pl.ds(base, per_w)], idx_v)

        def gat(i, b):
            return pltpu.make_async_copy(
                table_hbm.at[idx_v.at[pl.ds(i * CHUNK, CHUNK)]],
                rows_v.at[b],
                sg[b],
            )

        def sto(i, b):
            return pltpu.make_async_copy(
                rows_v.at[b],
                out_hbm.at[pl.ds(base + i * CHUNK, CHUNK)],
                so[b],
            )

        gat(0, 0).start()

        def body(j, carry):
            i0 = 2 * j
            i1 = i0 + 1
            gat(i0, 0).wait()
            gat(i1, 1).start()
            sto(i0, 0).start()

            @pl.when(j > 0)
            def _():
                sto(i0 - 1, 1).wait()

            gat(i1, 1).wait()
            sto(i1, 1).start()

            @pl.when(j < n2 - 1)
            def _():
                sto(i0, 0).wait()
                gat(i0 + 2, 0).start()

            return carry

        lax.fori_loop(0, n2, body, 0)
        sto(n_chunks - 2, 0).wait()
        sto(n_chunks - 1, 1).wait()

    return gather_kernel


def kernel(x, table):
    n_rows = x.shape[0] * x.shape[1]
    vocab, dim = table.shape
    fn = _build(n_rows, vocab, dim)
    out = fn(x.reshape(-1).astype(jnp.int32), table)
    return out.reshape(x.shape + (dim,))
